# baseline (speedup 1.0000x reference)
import jax
import jax.numpy as jnp
from jax import lax
from jax.experimental import pallas as pl
from jax.experimental.pallas import tpu as pltpu
from jax.experimental.pallas import tpu_sc as plsc

N_ROWS = 100000
B = 16384
NC, NS, L = 2, 16, 16
NW = NC * NS
BPW = B // NW
NCHUNK = BPW // 128
NGROUP = BPW // L

_A_COEF = (1.0, -1.0 / 6.0, 1.0 / 120.0, -1.0 / 5040.0,
           1.0 / 362880.0, -1.0 / 39916800.0)
_B_COEF = (0.5, -1.0 / 24.0, 1.0 / 720.0, -1.0 / 40320.0,
           1.0 / 3628800.0, -1.0 / 479001600.0)


def _poly(t, coef):
    acc = jnp.full((L,), coef[-1], jnp.float32)
    for c in reversed(coef[:-1]):
        acc = acc * t + c
    return acc


def _body(idx_hbm, rot_hbm, pw_hbm, sh_hbm, outr_hbm, outsh_hbm, *refs):
    idx_v = refs[0]
    ridx = refs[1:10]
    pidx = refs[10:13]
    oidx = refs[13:22]
    sidx = refs[22:24]
    soidx = refs[24:26]
    rcol = refs[26:35]
    wcol = refs[35:38]
    ocol = refs[38:47]
    scol = refs[47:49]
    sem = refs[49]

    wid = lax.axis_index("s") * NC + lax.axis_index("c")
    base = wid * BPW

    pltpu.sync_copy(idx_hbm.at[pl.ds(wid * NCHUNK, NCHUNK)], idx_v)

    lane = lax.iota(jnp.int32, L)
    lane9 = lane * 9
    lane2 = lane * 2

    for c in range(NCHUNK):
        obase = (base + c * 128) * 9
        obase2 = (base + c * 128) * 2

        def mk(k, carry):
            sl = pl.ds(k * L, L)
            i16 = idx_v[c, sl]
            i9 = i16 * 9
            i3 = i16 * 3
            i2 = i16 * 2
            opos = obase + k * (L * 9) + lane9
            opos2 = obase2 + k * (L * 2) + lane2
            for d in range(9):
                ridx[d][c, sl] = i9 + d
                oidx[d][c, sl] = opos + d
            for d in range(3):
                pidx[d][c, sl] = i3 + d
            for d in range(2):
                sidx[d][c, sl] = i2 + d
                soidx[d][c, sl] = opos2 + d
            return carry

        lax.fori_loop(0, 128 // L, mk, 0, unroll=False)

    copies = []
    for c in range(NCHUNK):
        sl = pl.ds(c * 128, 128)
        for d in range(2):
            copies.append(pltpu.async_copy(
                sh_hbm.at[sidx[d].at[c]], scol[d].at[sl], sem))
        for d in range(9):
            copies.append(pltpu.async_copy(
                rot_hbm.at[ridx[d].at[c]], rcol[d].at[sl], sem))
        for d in range(3):
            copies.append(pltpu.async_copy(
                pw_hbm.at[pidx[d].at[c]], wcol[d].at[sl], sem))
    for cp in copies:
        cp.wait()

    def group(g, carry):
        sl = pl.ds(g * L, L)
        w0 = wcol[0][sl]
        w1 = wcol[1][sl]
        w2 = wcol[2][sl]
        r = [rcol[d][sl] for d in range(9)]

        w00, w11, w22 = w0 * w0, w1 * w1, w2 * w2
        t = w00 + w11 + w22
        A = _poly(t, _A_COEF)
        Bc = _poly(t, _B_COEF)
        w01, w02, w12 = w0 * w1, w0 * w2, w1 * w2
        a0, a1, a2 = A * w0, A * w1, A * w2
        p00 = 1.0 - Bc * (w11 + w22)
        p01 = Bc * w01 - a2
        p02 = Bc * w02 + a1
        p10 = Bc * w01 + a2
        p11 = 1.0 - Bc * (w00 + w22)
        p12 = Bc * w12 - a0
        p20 = Bc * w02 - a1
        p21 = Bc * w12 + a0
        p22 = 1.0 - Bc * (w00 + w11)
        p = ((p00, p01, p02), (p10, p11, p12), (p20, p21, p22))
        for i in range(3):
            for j in range(3):
                acc = p[i][0] * r[0 * 3 + j]
                acc = acc + p[i][1] * r[1 * 3 + j]
                acc = acc + p[i][2] * r[2 * 3 + j]
                ocol[i * 3 + j][sl] = acc
        return carry

    lax.fori_loop(0, NGROUP, group, 0, unroll=False)

    copies = []
    for c in range(NCHUNK):
        sl = pl.ds(c * 128, 128)
        for d in range(9):
            copies.append(pltpu.async_copy(
                ocol[d].at[sl], outr_hbm.at[oidx[d].at[c]], sem))
        for d in range(2):
            copies.append(pltpu.async_copy(
                scol[d].at[sl], outsh_hbm.at[soidx[d].at[c]], sem))
    for cp in copies:
        cp.wait()


@jax.jit
def _run(idx2d, rot1d, pw1d, sh):
    mesh = plsc.VectorSubcoreMesh(core_axis_name="c", subcore_axis_name="s",
                                  num_cores=NC, num_subcores=NS)
    scratch = [pltpu.VMEM((NCHUNK, 128), jnp.int32)]
    scratch += [pltpu.VMEM((NCHUNK, 128), jnp.int32)] * 9
    scratch += [pltpu.VMEM((NCHUNK, 128), jnp.int32)] * 3
    scratch += [pltpu.VMEM((NCHUNK, 128), jnp.int32)] * 9
    scratch += [pltpu.VMEM((NCHUNK, 128), jnp.int32)] * 4
    scratch += [pltpu.VMEM((BPW,), jnp.float32)] * 9
    scratch += [pltpu.VMEM((BPW,), jnp.float32)] * 3
    scratch += [pltpu.VMEM((BPW,), jnp.float32)] * 9
    scratch += [pltpu.VMEM((BPW,), jnp.float32)] * 2
    scratch += [pltpu.SemaphoreType.DMA]
    f = pl.kernel(
        _body,
        out_type=(jax.ShapeDtypeStruct((B * 9,), jnp.float32),
                  jax.ShapeDtypeStruct((B * 2,), jnp.float32)),
        mesh=mesh,
        scratch_types=scratch,
    )
    return f(idx2d, rot1d, pw1d, sh)


def kernel(idx, rotations, perturbations_w, shifts):
    idx2d = idx.astype(jnp.int32).reshape(B // 128, 128)
    rot1d = rotations.reshape(N_ROWS * 9)
    pw1d = perturbations_w.reshape(N_ROWS * 3)
    sh1d = shifts.reshape(N_ROWS * 2)
    outr, outsh = _run(idx2d, rot1d, pw1d, sh1d)
    return outr.reshape(B, 3, 3), outsh.reshape(B, 2)


# speedup vs baseline: 15.1061x; 15.1061x over previous
import jax
import jax.numpy as jnp
from jax import lax
from jax.experimental import pallas as pl
from jax.experimental.pallas import tpu as pltpu
from jax.experimental.pallas import tpu_sc as plsc

N_ROWS = 100000
B = 16384
NC, NS, L = 2, 16, 16
NW = NC * NS
BPW = B // NW
NCHUNK = BPW // 128
NGROUP = BPW // L

_A_COEF = (1.0, -1.0 / 6.0, 1.0 / 120.0, -1.0 / 5040.0,
           1.0 / 362880.0, -1.0 / 39916800.0)
_B_COEF = (0.5, -1.0 / 24.0, 1.0 / 720.0, -1.0 / 40320.0,
           1.0 / 3628800.0, -1.0 / 479001600.0)


def _poly(t, coef):
    acc = jnp.full((L,), coef[-1], jnp.float32)
    for c in reversed(coef[:-1]):
        acc = acc * t + c
    return acc


def _body(idx_hbm, *refs):
    rin = refs[0:9]
    win = refs[9:12]
    sin_ = refs[12:14]
    rout = refs[14:23]
    sout = refs[23:25]
    idx_v = refs[25]
    rcol = refs[26:35]
    wcol = refs[35:38]
    scol = refs[38:40]
    ocol = refs[40:49]
    sem = refs[49]

    wid = lax.axis_index("s") * NC + lax.axis_index("c")
    base = wid * BPW

    pltpu.sync_copy(idx_hbm.at[pl.ds(wid * NCHUNK, NCHUNK)], idx_v)

    copies = []
    for c in range(NCHUNK):
        sl = pl.ds(c * 128, 128)
        ic = idx_v.at[c]
        for d in range(9):
            copies.append(pltpu.async_copy(rin[d].at[ic], rcol[d].at[sl], sem))
        for d in range(3):
            copies.append(pltpu.async_copy(win[d].at[ic], wcol[d].at[sl], sem))
        for d in range(2):
            copies.append(pltpu.async_copy(sin_[d].at[ic], scol[d].at[sl], sem))
    for cp in copies:
        cp.wait()

    def group(g, carry):
        sl = pl.ds(g * L, L)
        w0 = wcol[0][sl]
        w1 = wcol[1][sl]
        w2 = wcol[2][sl]
        r = [rcol[d][sl] for d in range(9)]

        w00, w11, w22 = w0 * w0, w1 * w1, w2 * w2
        t = w00 + w11 + w22
        A = _poly(t, _A_COEF)
        Bc = _poly(t, _B_COEF)
        w01, w02, w12 = w0 * w1, w0 * w2, w1 * w2
        a0, a1, a2 = A * w0, A * w1, A * w2
        p00 = 1.0 - Bc * (w11 + w22)
        p01 = Bc * w01 - a2
        p02 = Bc * w02 + a1
        p10 = Bc * w01 + a2
        p11 = 1.0 - Bc * (w00 + w22)
        p12 = Bc * w12 - a0
        p20 = Bc * w02 - a1
        p21 = Bc * w12 + a0
        p22 = 1.0 - Bc * (w00 + w11)
        p = ((p00, p01, p02), (p10, p11, p12), (p20, p21, p22))
        for i in range(3):
            for j in range(3):
                acc = p[i][0] * r[0 * 3 + j]
                acc = acc + p[i][1] * r[1 * 3 + j]
                acc = acc + p[i][2] * r[2 * 3 + j]
                ocol[i * 3 + j][sl] = acc
        return carry

    lax.fori_loop(0, NGROUP, group, 0, unroll=False)

    for d in range(9):
        pltpu.sync_copy(ocol[d], rout[d].at[pl.ds(base, BPW)])
    for d in range(2):
        pltpu.sync_copy(scol[d], sout[d].at[pl.ds(base, BPW)])


@jax.jit
def _run(idx2d, *planes):
    mesh = plsc.VectorSubcoreMesh(core_axis_name="c", subcore_axis_name="s",
                                  num_cores=NC, num_subcores=NS)
    scratch = [pltpu.VMEM((NCHUNK, 128), jnp.int32)]
    scratch += [pltpu.VMEM((BPW,), jnp.float32)] * 23
    scratch += [pltpu.SemaphoreType.DMA]
    f = pl.kernel(
        _body,
        out_type=tuple([jax.ShapeDtypeStruct((B,), jnp.float32)] * 11),
        mesh=mesh,
        scratch_types=scratch,
    )
    return f(idx2d, *planes)


def kernel(idx, rotations, perturbations_w, shifts):
    idx2d = idx.astype(jnp.int32).reshape(B // 128, 128)
    planes = [rotations[:, i, j] for i in range(3) for j in range(3)]
    planes += [perturbations_w[:, c] for c in range(3)]
    planes += [shifts[:, c] for c in range(2)]
    outs = _run(idx2d, *planes)
    rots = jnp.stack(outs[0:9], axis=-1).reshape(B, 3, 3)
    sh = jnp.stack(outs[9:11], axis=-1)
    return rots, sh
